# baseline (speedup 1.0000x reference)
import jax
import jax.numpy as jnp
from jax.experimental import pallas as pl
from jax.experimental.pallas import tpu as pltpu


def _fused_kernel(x_ref, w_ref, b_ref, g1_ref, g2_ref, out_ref, support_ref):
    @pl.when(pl.program_id(0) == 0)
    def _compute_support():
        support_ref[...] = (
            jnp.dot(x_ref[...], w_ref[...], preferred_element_type=jnp.float32)
            + b_ref[...]
        )

    nh = support_ref.shape[0] // 2
    g1 = g1_ref[:, 0, 0, :]
    g2 = g2_ref[:, 0, 0, :]
    out_ref[...] = jnp.dot(
        g1, support_ref[:nh, :], preferred_element_type=jnp.float32
    ) + jnp.dot(g2, support_ref[nh:, :], preferred_element_type=jnp.float32)


def kernel(input, G, W, b):
    n, d_in = input.shape
    d_out = W.shape[1]
    m = G.shape[0]
    bm = 400
    nh = n // 2
    grid = (m // bm,)
    g4 = G.reshape(m, 2, 1, nh)
    return pl.pallas_call(
        _fused_kernel,
        grid=grid,
        in_specs=[
            pl.BlockSpec((n, d_in), lambda i: (0, 0)),
            pl.BlockSpec((d_in, d_out), lambda i: (0, 0)),
            pl.BlockSpec((1, d_out), lambda i: (0, 0)),
            pl.BlockSpec((bm, 1, 1, nh), lambda i: (i, 0, 0, 0)),
            pl.BlockSpec((bm, 1, 1, nh), lambda i: (i, 1, 0, 0)),
        ],
        out_specs=pl.BlockSpec((bm, d_out), lambda i: (i, 0)),
        out_shape=jax.ShapeDtypeStruct((m, d_out), jnp.float32),
        scratch_shapes=[pltpu.VMEM((n, d_out), jnp.float32)],
    )(input, W, b.reshape(1, d_out), g4, g4)


# speedup vs baseline: 23.4801x; 23.4801x over previous
import jax
import jax.numpy as jnp
from jax.experimental import pallas as pl
from jax.experimental.pallas import tpu as pltpu


def _fused_kernel(x_ref, w_ref, b_ref, ga_ref, gb_ref, out_ref, support_ref):
    @pl.when(pl.program_id(0) == 0)
    def _compute_support():
        support_ref[...] = (
            jnp.dot(x_ref[...], w_ref[...], preferred_element_type=jnp.float32)
            + b_ref[...]
        )

    bm = ga_ref.shape[0]
    out_ref[:bm, :] = jnp.dot(
        ga_ref[...], support_ref[...], preferred_element_type=jnp.float32
    )
    out_ref[bm:, :] = jnp.dot(
        gb_ref[...], support_ref[...], preferred_element_type=jnp.float32
    )


def kernel(input, G, W, b):
    n, d_in = input.shape
    d_out = W.shape[1]
    m = G.shape[0]
    bm = 200
    grid = (m // (2 * bm),)
    return pl.pallas_call(
        _fused_kernel,
        grid=grid,
        in_specs=[
            pl.BlockSpec((n, d_in), lambda i: (0, 0)),
            pl.BlockSpec((d_in, d_out), lambda i: (0, 0)),
            pl.BlockSpec((1, d_out), lambda i: (0, 0)),
            pl.BlockSpec((bm, n), lambda i: (2 * i, 0)),
            pl.BlockSpec((bm, n), lambda i: (2 * i + 1, 0)),
        ],
        out_specs=pl.BlockSpec((2 * bm, d_out), lambda i: (i, 0)),
        out_shape=jax.ShapeDtypeStruct((m, d_out), jnp.float32),
        scratch_shapes=[pltpu.VMEM((n, d_out), jnp.float32)],
    )(input, W, b.reshape(1, d_out), G, G)
